# baseline (speedup 1.0000x reference)
import functools

import jax
import jax.numpy as jnp
from jax.experimental import pallas as pl


def _fwd(x_ref, att_ref, ws_ref, bs_ref, wd_ref, bd_ref, a_ref, wg_ref,
         bg_ref, lng_ref, lnb_ref, wfc_ref, bfc_ref, ln2g_ref, ln2b_ref,
         o_ref, *, k, slots):
    x = x_ref[0]
    att = att_ref[0]
    L = att.shape[0]
    f32 = jnp.float32

    fs = jnp.dot(x, ws_ref[...], preferred_element_type=f32) + bs_ref[...]
    fd = jnp.dot(x, wd_ref[...], preferred_element_type=f32) + bd_ref[...]
    fs_b = fs.astype(jnp.bfloat16)

    lane_iota = jax.lax.broadcasted_iota(jnp.int32, (L, L), 1).astype(f32)
    slot_iota = jax.lax.broadcasted_iota(jnp.int32, (L, slots), 1).astype(f32)
    neg_inf = jnp.float32(-jnp.inf)

    work = att
    vals = jnp.zeros((L, slots), f32)
    idxs = jnp.zeros((L, slots), f32)
    scores = jnp.zeros((L, slots), f32)
    for t in range(k):
        m = jnp.max(work, axis=1, keepdims=True)
        jstar = jnp.argmax(work, axis=1, keepdims=True).astype(f32)
        onehot = lane_iota == jstar
        work = jnp.where(onehot, neg_inf, work)
        vals = jnp.where(slot_iota == t, m, vals)
        idxs = jnp.where(slot_iota == t, jstar, idxs)
        g = jnp.dot(onehot.astype(jnp.bfloat16), fs_b,
                    preferred_element_type=f32)
        e = g + fd
        e = jnp.maximum(e, 0.2 * e)
        s = jnp.sum(e * a_ref[...], axis=1, keepdims=True)
        scores = jnp.where(slot_iota == t, s, scores)

    sc = jnp.where(slot_iota >= k, neg_inf,
                   jnp.where(vals > 0.0, scores, jnp.float32(-1e9)))
    mx = jnp.max(sc, axis=1, keepdims=True)
    ex = jnp.exp(sc - mx)
    alpha = ex / jnp.sum(ex, axis=1, keepdims=True)

    P = jnp.zeros((L, L), f32)
    for t in range(k):
        a_col = jax.lax.slice_in_dim(alpha, t, t + 1, axis=1)
        i_col = jax.lax.slice_in_dim(idxs, t, t + 1, axis=1)
        P = P + jnp.where(lane_iota == i_col, a_col, 0.0)
    rst = jnp.dot(P, fs, preferred_element_type=f32)

    h = rst + x
    h = jnp.maximum(h, 0.01 * h)

    z = jnp.sum(h * wg_ref[...], axis=1, keepdims=True) + bg_ref[...]
    zm = jnp.max(z, axis=0, keepdims=True)
    ze = jnp.exp(z - zm)
    gate = ze / jnp.sum(ze, axis=0, keepdims=True)
    pooled = jnp.sum(gate * h, axis=0, keepdims=True)
    pooled = jnp.maximum(pooled, 0.0)

    mu = jnp.mean(pooled, axis=1, keepdims=True)
    var = jnp.mean((pooled - mu) ** 2, axis=1, keepdims=True)
    pn = (pooled - mu) / jnp.sqrt(var + 1e-5) * lng_ref[...] + lnb_ref[...]

    y = jnp.dot(pn, wfc_ref[...], preferred_element_type=f32) + bfc_ref[...]
    mu2 = jnp.mean(y, axis=1, keepdims=True)
    var2 = jnp.mean((y - mu2) ** 2, axis=1, keepdims=True)
    o_ref[0] = ((y - mu2) / jnp.sqrt(var2 + 1e-5) * ln2g_ref[...]
                + ln2b_ref[...])


def kernel(hidden_state, attention, mask, encoded_inputs, W_src, b_src,
           W_dst, b_dst, attn_vec, W_gate, b_gate, ln_g, ln_b, W_fc, b_fc,
           ln2_g, ln2_b):
    B, L, D = hidden_state.shape
    D_out = W_fc.shape[1]
    k = int(round(0.05 * L))
    slots = 32

    full = lambda shape: pl.BlockSpec(shape, lambda b: (0,) * len(shape))
    out = pl.pallas_call(
        functools.partial(_fwd, k=k, slots=slots),
        grid=(B,),
        in_specs=[
            pl.BlockSpec((1, L, D), lambda b: (b, 0, 0)),
            pl.BlockSpec((1, L, L), lambda b: (b, 0, 0)),
            full((D, D)),
            full((1, D)),
            full((D, D)),
            full((1, D)),
            full((1, D)),
            full((1, D)),
            full((1, 1)),
            full((1, D)),
            full((1, D)),
            full((D, D_out)),
            full((1, D_out)),
            full((1, D_out)),
            full((1, D_out)),
        ],
        out_specs=pl.BlockSpec((1, 1, D_out), lambda b: (b, 0, 0)),
        out_shape=jax.ShapeDtypeStruct((B, 1, D_out), jnp.float32),
    )(
        hidden_state, attention, W_src, b_src.reshape(1, D), W_dst,
        b_dst.reshape(1, D), attn_vec.reshape(1, D), W_gate.reshape(1, D),
        b_gate.reshape(1, 1), ln_g.reshape(1, D), ln_b.reshape(1, D),
        W_fc, b_fc.reshape(1, D_out), ln2_g.reshape(1, D_out),
        ln2_b.reshape(1, D_out),
    )
    return out.reshape(B, D_out)


# speedup vs baseline: 1.7927x; 1.7927x over previous
import functools

import jax
import jax.numpy as jnp
from jax.experimental import pallas as pl


def _fwd(x_ref, att_ref, ws_ref, bs_ref, wd_ref, bd_ref, a_ref, wg_ref,
         bg_ref, lng_ref, lnb_ref, wfc_ref, bfc_ref, ln2g_ref, ln2b_ref,
         o_ref, *, k, slots):
    x = x_ref[0]
    att = att_ref[0]
    L = att.shape[0]
    f32 = jnp.float32

    fs = jnp.dot(x, ws_ref[...], preferred_element_type=f32) + bs_ref[...]
    fd = jnp.dot(x, wd_ref[...], preferred_element_type=f32) + bd_ref[...]
    fs_b = fs.astype(jnp.bfloat16)

    lane_iota = jax.lax.broadcasted_iota(jnp.int32, (L, L), 1).astype(f32)
    slot_iota = jax.lax.broadcasted_iota(jnp.int32, (L, slots), 1).astype(f32)
    neg_inf = jnp.float32(-jnp.inf)

    work = att
    vals = jnp.zeros((L, slots), f32)
    idxs = jnp.zeros((L, slots), f32)
    scores = jnp.zeros((L, slots), f32)
    for t in range(k):
        m = jnp.max(work, axis=1, keepdims=True)
        jstar = jnp.min(jnp.where(work == m, lane_iota, jnp.float32(L)),
                        axis=1, keepdims=True)
        onehot = lane_iota == jstar
        work = jnp.where(onehot, neg_inf, work)
        vals = jnp.where(slot_iota == t, m, vals)
        idxs = jnp.where(slot_iota == t, jstar, idxs)
        g = jnp.dot(onehot.astype(jnp.bfloat16), fs_b,
                    preferred_element_type=f32)
        e = g + fd
        e = jnp.maximum(e, 0.2 * e)
        s = jnp.sum(e * a_ref[...], axis=1, keepdims=True)
        scores = jnp.where(slot_iota == t, s, scores)

    sc = jnp.where(slot_iota >= k, neg_inf,
                   jnp.where(vals > 0.0, scores, jnp.float32(-1e9)))
    mx = jnp.max(sc, axis=1, keepdims=True)
    ex = jnp.exp(sc - mx)
    alpha = ex / jnp.sum(ex, axis=1, keepdims=True)

    P = jnp.zeros((L, L), f32)
    for t in range(k):
        a_col = jax.lax.slice_in_dim(alpha, t, t + 1, axis=1)
        i_col = jax.lax.slice_in_dim(idxs, t, t + 1, axis=1)
        P = P + jnp.where(lane_iota == i_col, a_col, 0.0)
    rst = jnp.dot(P, fs, preferred_element_type=f32)

    h = rst + x
    h = jnp.maximum(h, 0.01 * h)

    z = jnp.sum(h * wg_ref[...], axis=1, keepdims=True) + bg_ref[...]
    zm = jnp.max(z, axis=0, keepdims=True)
    ze = jnp.exp(z - zm)
    gate = ze / jnp.sum(ze, axis=0, keepdims=True)
    pooled = jnp.sum(gate * h, axis=0, keepdims=True)
    pooled = jnp.maximum(pooled, 0.0)

    mu = jnp.mean(pooled, axis=1, keepdims=True)
    var = jnp.mean((pooled - mu) ** 2, axis=1, keepdims=True)
    pn = (pooled - mu) / jnp.sqrt(var + 1e-5) * lng_ref[...] + lnb_ref[...]

    y = jnp.dot(pn, wfc_ref[...], preferred_element_type=f32) + bfc_ref[...]
    mu2 = jnp.mean(y, axis=1, keepdims=True)
    var2 = jnp.mean((y - mu2) ** 2, axis=1, keepdims=True)
    o_ref[0] = ((y - mu2) / jnp.sqrt(var2 + 1e-5) * ln2g_ref[...]
                + ln2b_ref[...])


def kernel(hidden_state, attention, mask, encoded_inputs, W_src, b_src,
           W_dst, b_dst, attn_vec, W_gate, b_gate, ln_g, ln_b, W_fc, b_fc,
           ln2_g, ln2_b):
    B, L, D = hidden_state.shape
    D_out = W_fc.shape[1]
    k = int(round(0.05 * L))
    slots = 32

    full = lambda shape: pl.BlockSpec(shape, lambda b: (0,) * len(shape))
    out = pl.pallas_call(
        functools.partial(_fwd, k=k, slots=slots),
        grid=(B,),
        in_specs=[
            pl.BlockSpec((1, L, D), lambda b: (b, 0, 0)),
            pl.BlockSpec((1, L, L), lambda b: (b, 0, 0)),
            full((D, D)),
            full((1, D)),
            full((D, D)),
            full((1, D)),
            full((1, D)),
            full((1, D)),
            full((1, 1)),
            full((1, D)),
            full((1, D)),
            full((D, D_out)),
            full((1, D_out)),
            full((1, D_out)),
            full((1, D_out)),
        ],
        out_specs=pl.BlockSpec((1, 1, D_out), lambda b: (b, 0, 0)),
        out_shape=jax.ShapeDtypeStruct((B, 1, D_out), jnp.float32),
    )(
        hidden_state, attention, W_src, b_src.reshape(1, D), W_dst,
        b_dst.reshape(1, D), attn_vec.reshape(1, D), W_gate.reshape(1, D),
        b_gate.reshape(1, 1), ln_g.reshape(1, D), ln_b.reshape(1, D),
        W_fc, b_fc.reshape(1, D_out), ln2_g.reshape(1, D_out),
        ln2_b.reshape(1, D_out),
    )
    return out.reshape(B, D_out)


# speedup vs baseline: 1.7943x; 1.0009x over previous
import functools

import jax
import jax.numpy as jnp
from jax.experimental import pallas as pl


def _fwd(x_ref, att_ref, ws_ref, bs_ref, wd_ref, bd_ref, a_ref, wg_ref,
         bg_ref, lng_ref, lnb_ref, wfc_ref, bfc_ref, ln2g_ref, ln2b_ref,
         o_ref, *, k, slots):
    x = x_ref[0]
    att = att_ref[0]
    L = att.shape[0]
    f32 = jnp.float32

    fs = jnp.dot(x, ws_ref[...], preferred_element_type=f32) + bs_ref[...]
    fd = jnp.dot(x, wd_ref[...], preferred_element_type=f32) + bd_ref[...]
    fs_b = fs.astype(jnp.bfloat16)

    lane_iota = jax.lax.broadcasted_iota(jnp.int32, (L, L), 1).astype(f32)
    slot_iota = jax.lax.broadcasted_iota(jnp.int32, (L, slots), 1).astype(f32)
    neg_inf = jnp.float32(-jnp.inf)

    work = att
    vals = jnp.zeros((L, slots), f32)
    idxs = jnp.zeros((L, slots), f32)
    scores = jnp.zeros((L, slots), f32)
    for t in range(k):
        m = jnp.max(work, axis=1, keepdims=True)
        jstar = jnp.min(jnp.where(work == m, lane_iota, jnp.float32(L)),
                        axis=1, keepdims=True)
        onehot = lane_iota == jstar
        work = jnp.where(onehot, neg_inf, work)
        vals = jnp.where(slot_iota == t, m, vals)
        idxs = jnp.where(slot_iota == t, jstar, idxs)
        g = jnp.dot(onehot.astype(jnp.bfloat16), fs_b,
                    preferred_element_type=f32)
        e = g + fd
        e = jnp.maximum(e, 0.2 * e)
        s = jnp.sum(e * a_ref[...], axis=1, keepdims=True)
        scores = jnp.where(slot_iota == t, s, scores)

    sc = jnp.where(slot_iota >= k, neg_inf,
                   jnp.where(vals > 0.0, scores, jnp.float32(-1e9)))
    mx = jnp.max(sc, axis=1, keepdims=True)
    ex = jnp.exp(sc - mx)
    alpha = ex / jnp.sum(ex, axis=1, keepdims=True)

    P = jnp.zeros((L, L), f32)
    for t in range(k):
        a_col = jax.lax.slice_in_dim(alpha, t, t + 1, axis=1)
        i_col = jax.lax.slice_in_dim(idxs, t, t + 1, axis=1)
        P = P + jnp.where(lane_iota == i_col, a_col, 0.0)
    rst = jnp.dot(P, fs, preferred_element_type=f32)

    h = rst + x
    h = jnp.maximum(h, 0.01 * h)

    z = jnp.sum(h * wg_ref[...], axis=1, keepdims=True) + bg_ref[...]
    zm = jnp.max(z, axis=0, keepdims=True)
    ze = jnp.exp(z - zm)
    gate = ze / jnp.sum(ze, axis=0, keepdims=True)
    pooled = jnp.sum(gate * h, axis=0, keepdims=True)
    pooled = jnp.maximum(pooled, 0.0)

    mu = jnp.mean(pooled, axis=1, keepdims=True)
    var = jnp.mean((pooled - mu) ** 2, axis=1, keepdims=True)
    pn = (pooled - mu) / jnp.sqrt(var + 1e-5) * lng_ref[...] + lnb_ref[...]

    y = jnp.dot(pn, wfc_ref[...], preferred_element_type=f32) + bfc_ref[...]
    mu2 = jnp.mean(y, axis=1, keepdims=True)
    var2 = jnp.mean((y - mu2) ** 2, axis=1, keepdims=True)
    o_ref[0] = ((y - mu2) / jnp.sqrt(var2 + 1e-5) * ln2g_ref[...]
                + ln2b_ref[...])


def kernel(hidden_state, attention, mask, encoded_inputs, W_src, b_src,
           W_dst, b_dst, attn_vec, W_gate, b_gate, ln_g, ln_b, W_fc, b_fc,
           ln2_g, ln2_b):
    B, L, D = hidden_state.shape
    D_out = W_fc.shape[1]
    k = int(round(0.05 * L))
    slots = max(32, ((k + 7) // 8) * 8)

    full = lambda shape: pl.BlockSpec(shape, lambda b: (0,) * len(shape))
    out = pl.pallas_call(
        functools.partial(_fwd, k=k, slots=slots),
        grid=(B,),
        in_specs=[
            pl.BlockSpec((1, L, D), lambda b: (b, 0, 0)),
            pl.BlockSpec((1, L, L), lambda b: (b, 0, 0)),
            full((D, D)),
            full((1, D)),
            full((D, D)),
            full((1, D)),
            full((1, D)),
            full((1, D)),
            full((1, 1)),
            full((1, D)),
            full((1, D)),
            full((D, D_out)),
            full((1, D_out)),
            full((1, D_out)),
            full((1, D_out)),
        ],
        out_specs=pl.BlockSpec((1, 1, D_out), lambda b: (b, 0, 0)),
        out_shape=jax.ShapeDtypeStruct((B, 1, D_out), jnp.float32),
    )(
        hidden_state, attention, W_src, b_src.reshape(1, D), W_dst,
        b_dst.reshape(1, D), attn_vec.reshape(1, D), W_gate.reshape(1, D),
        b_gate.reshape(1, 1), ln_g.reshape(1, D), ln_b.reshape(1, D),
        W_fc, b_fc.reshape(1, D_out), ln2_g.reshape(1, D_out),
        ln2_b.reshape(1, D_out),
    )
    return out.reshape(B, D_out)
